# baseline (speedup 1.0000x reference)
import functools

import jax
import jax.numpy as jnp
from jax import lax
from jax.experimental import pallas as pl
from jax.experimental.pallas import tpu as pltpu
from jax.experimental.pallas import tpu_sc as plsc

N = 10000
E = 160000
NP = 10240
W_WIN = 160
NC = 2
NS = 16
N_WIN = 34
EP = NC * NS * N_WIN * W_WIN
EDGE_WINDOWS = E // W_WIN
ROWS_PER_TILE = NP // NS
_CHUNK = 128


def _take16(x, idx):
    return lax.gather(
        x, idx[:, None],
        dimension_numbers=lax.GatherDimensionNumbers(
            offset_dims=(), collapsed_slice_dims=(0,), start_index_map=(0,)),
        slice_sizes=(1,),
        mode=lax.GatherScatterMode.PROMISE_IN_BOUNDS)


def _edge_sc_kernel(width):
    mesh = plsc.VectorSubcoreMesh(core_axis_name="c", subcore_axis_name="s")

    @functools.partial(
        pl.kernel,
        mesh=mesh,
        compiler_params=pltpu.CompilerParams(use_tc_tiling_on_sc=False),
        out_type=jax.ShapeDtypeStruct((NC, NP, width), jnp.float32),
        scratch_types=[
            pltpu.VMEM((2, 2, W_WIN), jnp.int32),
            pltpu.VMEM((2, 2, W_WIN // 2), jnp.int32),
            pltpu.VMEM((2, W_WIN, 16), jnp.float32),
            pltpu.VMEM((2, W_WIN, width), jnp.float32),
            pltpu.VMEM((2, W_WIN, width), jnp.float32),
            pltpu.VMEM_SHARED((NP, width), jnp.float32),
            pltpu.SemaphoreType.DMA((2,)),
            pltpu.SemaphoreType.DMA((2,)),
            pltpu.SemaphoreType.DMA((2,)),
            pltpu.SemaphoreType.DMA((2,)),
        ],
    )
    def k(ei_h, tq_h, tkv_h, out_h, idxb, sidx, qr, kvr, upd, acc,
          sem_i, sem_q, sem_kv, sem_s):
        cid = lax.axis_index("c")
        sid = lax.axis_index("s")
        wid = cid * NS + sid
        r0 = sid * ROWS_PER_TILE
        H = W_WIN // 2

        zero16 = jnp.zeros((16,), jnp.float32)

        def zr(i, c):
            for j in range(width // 16):
                upd[0, i, pl.ds(16 * j, 16)] = zero16
            return c

        lax.fori_loop(0, _CHUNK, zr, 0)
        for j in range(ROWS_PER_TILE // _CHUNK):
            pltpu.sync_copy(upd.at[0, pl.ds(0, _CHUNK)],
                            acc.at[pl.ds(r0 + j * _CHUNK, _CHUNK)])
        plsc.subcore_barrier()

        lane = lax.iota(jnp.int32, 16)
        eight = jnp.full((16,), 8, jnp.int32)

        def start_idx(w, p):
            g = wid * N_WIN + w

            @pl.when(g < EDGE_WINDOWS)
            def _():
                off = g * W_WIN
                pltpu.async_copy(ei_h.at[0, pl.ds(off, W_WIN)],
                                 idxb.at[p, 0], sem_i.at[p])
                pltpu.async_copy(ei_h.at[1, pl.ds(off, W_WIN)],
                                 idxb.at[p, 1], sem_i.at[p])

        def ready_idx(w, p):
            g = wid * N_WIN + w

            @pl.when(g < EDGE_WINDOWS)
            def _():
                for r in range(2):
                    pltpu.make_async_copy(ei_h.at[r, pl.ds(0, W_WIN)],
                                          idxb.at[p, r], sem_i.at[p]).wait()

            @pl.when(g >= EDGE_WINDOWS)
            def _():
                for kk in range(W_WIN // 16):
                    ev = g * W_WIN + 16 * kk + lane
                    v = jnp.where(ev < E + N, ev - E, 10000 + (ev & 63))
                    idxb[p, 0, pl.ds(16 * kk, 16)] = v
                    idxb[p, 1, pl.ds(16 * kk, 16)] = v

        def start_gathers(w, p):
            for s in range(2):
                pltpu.async_copy(tq_h.at[idxb.at[p, 0, pl.ds(s * H, H)]],
                                 qr.at[p, pl.ds(s * H, H)], sem_q.at[p])
                pltpu.async_copy(tkv_h.at[idxb.at[p, 1, pl.ds(s * H, H)]],
                                 kvr.at[p, pl.ds(s * H, H)], sem_kv.at[p])

        def wait_gathers(p):
            for s in range(2):
                pltpu.make_async_copy(tq_h.at[idxb.at[p, 0, pl.ds(s * H, H)]],
                                      qr.at[p, pl.ds(s * H, H)],
                                      sem_q.at[p]).wait()
                pltpu.make_async_copy(tkv_h.at[idxb.at[p, 1, pl.ds(s * H, H)]],
                                      kvr.at[p, pl.ds(s * H, H)],
                                      sem_kv.at[p]).wait()

        def start_scatter(p):
            for s in range(2):
                pltpu.async_copy(upd.at[p, pl.ds(s * H, H)],
                                 acc.at[sidx.at[p, s]], sem_s.at[p],
                                 add=True)

        def wait_scatter(p):
            for s in range(2):
                pltpu.make_async_copy(upd.at[p, pl.ds(s * H, H)],
                                      acc.at[sidx.at[p, s]],
                                      sem_s.at[p]).wait()

        def compute(p):
            def edge(e, c2):
                for u in range(4):
                    eu = 4 * e + u
                    q = qr[p, eu]
                    if width == 80:
                        kvk = kvr[p, eu, pl.ds(64, 16)]
                        ex = jnp.exp(q * kvk)
                        upd[p, eu, pl.ds(64, 16)] = ex
                        for j in range(4):
                            bj = _take16(ex, (lane >> 3) + 2 * j)
                            vj = kvr[p, eu, pl.ds(16 * j, 16)]
                            upd[p, eu, pl.ds(16 * j, 16)] = bj * vj
                    else:
                        kv = kvr[p, eu]
                        ex = jnp.exp(q * kv)
                        b0v = _take16(ex, eight)
                        upd[p, eu] = b0v * kv
                return c2

            lax.fori_loop(0, W_WIN // 4, edge, 0)

        def body(w, p):
            @pl.when(w >= 2)
            def _():
                wait_scatter(p)

            wait_gathers(p)
            for s in range(2):
                for i in range(H // 16):
                    sidx[p, s, pl.ds(16 * i, 16)] = (
                        idxb[p, 0, pl.ds(s * H + 16 * i, 16)])

            @pl.when(w + 1 < N_WIN)
            def _():
                ready_idx(w + 1, 1 - p)
                start_gathers(w + 1, 1 - p)

            @pl.when(w + 2 < N_WIN)
            def _():
                start_idx(w + 2, p)

            compute(p)
            start_scatter(p)

        start_idx(0, 0)
        start_idx(1, 1)
        ready_idx(0, 0)
        start_gathers(0, 0)

        def outer(g, carry):
            body(2 * g, 0)
            body(2 * g + 1, 1)
            return carry

        lax.fori_loop(0, N_WIN // 2, outer, 0)
        if N_WIN % 2:
            body(N_WIN - 1, 0)
        wait_scatter(0)
        wait_scatter(1)
        plsc.subcore_barrier()

        for j in range(ROWS_PER_TILE // _CHUNK):
            sl = pl.ds(r0 + j * _CHUNK, _CHUNK)
            pltpu.sync_copy(acc.at[sl], upd.at[0, pl.ds(0, _CHUNK)])
            pltpu.sync_copy(upd.at[0, pl.ds(0, _CHUNK)], out_h.at[cid, sl])

    return k


_edge_sc_80 = _edge_sc_kernel(80)
_edge_sc_16 = _edge_sc_kernel(16)

_HIGH = None
_HIGHEST = lax.Precision.HIGHEST
def _prep_kernel(x_ref, wq_ref, bq_ref, wkv_ref, bkv_ref, tq_ref, tkv_ref):
    x = x_ref[...]
    tq_ref[...] = jax.nn.relu(
        jnp.dot(x, wq_ref[...], preferred_element_type=jnp.float32,
                precision=None) + bq_ref[...])
    t = jnp.dot(x, wkv_ref[...], preferred_element_type=jnp.float32,
                precision=None) + bkv_ref[...]
    ci = lax.broadcasted_iota(jnp.int32, t.shape, 1)
    tkv_ref[...] = jnp.where(ci >= 64, jax.nn.relu(t), t)


def _mid_kernel(a_ref, b0_ref, wq1_ref, bq1_ref, wkv1_ref, bkv1_ref,
                tq1_ref, tkv1_ref):
    s = a_ref[0] + a_ref[1]
    d8 = s[:, 64:72]
    r = lax.broadcasted_iota(jnp.int32, (8, 64), 0)
    c = lax.broadcasted_iota(jnp.int32, (8, 64), 1)
    onehot = (r == c // 8).astype(jnp.float32)
    d64 = jnp.dot(d8, onehot, preferred_element_type=jnp.float32,
                  precision=_HIGHEST)
    h = jax.nn.relu(s[:, 0:64] / (d64 + 1e-16) + b0_ref[...])
    tq1_ref[...] = jax.nn.relu(
        jnp.dot(h, wq1_ref[...], preferred_element_type=jnp.float32,
                precision=_HIGHEST) + bq1_ref[...])
    t = jnp.dot(h, wkv1_ref[...], preferred_element_type=jnp.float32,
                precision=_HIGHEST) + bkv1_ref[...]
    ci = lax.broadcasted_iota(jnp.int32, t.shape, 1)
    tkv1_ref[...] = jnp.where(ci >= 7, jax.nn.relu(t), t)


def _final_kernel(a_ref, b1_ref, out_ref):
    s = a_ref[0] + a_ref[1]
    r = lax.broadcasted_iota(jnp.int32, (128, 128), 0)
    c = lax.broadcasted_iota(jnp.int32, (128, 128), 1)
    md = (r == (c // 16) * 16 + 7).astype(jnp.float32)
    dvec = jnp.dot(s, md, preferred_element_type=jnp.float32,
                   precision=_HIGHEST)
    out_ref[...] = s / (dvec + 1e-16) + b1_ref[...]


def kernel(x, edge_index, Wq0, bq0, Wk0, bk0, W0, b0, Wq1, bq1, Wk1, bk1,
           W1, b1):
    f32 = jnp.float32
    wq16 = jnp.concatenate([Wq0, jnp.zeros((256, 8), f32)], 1)
    bq16 = jnp.concatenate([bq0, jnp.zeros((8,), f32)]).reshape(1, 16)
    wkv80 = jnp.concatenate([W0, Wk0, jnp.zeros((256, 8), f32)], 1)
    bkv80 = jnp.concatenate(
        [jnp.zeros((64,), f32), bk0, jnp.zeros((8,), f32)]).reshape(1, 80)

    wq1p = jnp.concatenate(
        [jnp.zeros((64, 8), f32), Wq1, jnp.zeros((64, 7), f32)], 1)
    bq1p = jnp.concatenate(
        [jnp.zeros((8,), f32), bq1, jnp.zeros((7,), f32)]).reshape(1, 16)
    wkv1p = jnp.concatenate(
        [W1, jnp.zeros((64, 1), f32), Wk1, jnp.zeros((64, 7), f32)], 1)
    bkv1p = jnp.concatenate(
        [jnp.zeros((7,), f32), jnp.ones((1,), f32), bk1,
         jnp.zeros((7,), f32)]).reshape(1, 16)
    b1p = jnp.tile(jnp.concatenate([b1, jnp.zeros((9,), f32)]),
                   8).reshape(1, 128)
    b0r = b0.reshape(1, 64)


    blk = 1000
    grid = N // blk

    tq0, tkv0 = pl.pallas_call(
        _prep_kernel,
        grid=(grid,),
        in_specs=[
            pl.BlockSpec((blk, 256), lambda i: (i, 0)),
            pl.BlockSpec((256, 16), lambda i: (0, 0)),
            pl.BlockSpec((1, 16), lambda i: (0, 0)),
            pl.BlockSpec((256, 80), lambda i: (0, 0)),
            pl.BlockSpec((1, 80), lambda i: (0, 0)),
        ],
        out_specs=[
            pl.BlockSpec((blk, 16), lambda i: (i, 0)),
            pl.BlockSpec((blk, 80), lambda i: (i, 0)),
        ],
        out_shape=[
            jax.ShapeDtypeStruct((NP, 16), f32),
            jax.ShapeDtypeStruct((NP, 80), f32),
        ],
    )(x, wq16, bq16, wkv80, bkv80)

    acc0 = _edge_sc_80(edge_index, tq0, tkv0)

    blk2 = 1024
    grid2 = NP // blk2
    tq1, tkv1 = pl.pallas_call(
        _mid_kernel,
        grid=(grid2,),
        in_specs=[
            pl.BlockSpec((2, blk2, 80), lambda i: (0, i, 0)),
            pl.BlockSpec((1, 64), lambda i: (0, 0)),
            pl.BlockSpec((64, 16), lambda i: (0, 0)),
            pl.BlockSpec((1, 16), lambda i: (0, 0)),
            pl.BlockSpec((64, 16), lambda i: (0, 0)),
            pl.BlockSpec((1, 16), lambda i: (0, 0)),
        ],
        out_specs=[
            pl.BlockSpec((blk2, 16), lambda i: (i, 0)),
            pl.BlockSpec((blk2, 16), lambda i: (i, 0)),
        ],
        out_shape=[
            jax.ShapeDtypeStruct((NP, 16), f32),
            jax.ShapeDtypeStruct((NP, 16), f32),
        ],
    )(acc0, b0r, wq1p, bq1p, wkv1p, bkv1p)

    acc1 = _edge_sc_16(edge_index, tq1, tkv1)

    npk = NP // 8
    acc1v = acc1.reshape(NC, npk, 128)
    blk3 = 128
    outp = pl.pallas_call(
        _final_kernel,
        grid=(npk // blk3,),
        in_specs=[
            pl.BlockSpec((2, blk3, 128), lambda i: (0, i, 0)),
            pl.BlockSpec((1, 128), lambda i: (0, 0)),
        ],
        out_specs=pl.BlockSpec((blk3, 128), lambda i: (i, 0)),
        out_shape=jax.ShapeDtypeStruct((npk, 128), f32),
    )(acc1v, b1p)

    return outp.reshape(NP, 16)[:N, :7]
